# baseline (speedup 1.0000x reference)
import functools

import jax
import jax.numpy as jnp
from jax import lax
from jax.experimental import pallas as pl
from jax.experimental.pallas import tpu as pltpu
from jax.experimental.pallas import tpu_sc as plsc

B = 16384
H = 200
E = 32
HID = 128
OUT = 2

NC, NS = 2, 16
NW = NC * NS
RPG = 128
G = B // RPG
GPW = G // NW
IPG = RPG * H

_mesh = plsc.VectorSubcoreMesh(
    core_axis_name="c", subcore_axis_name="s", num_cores=NC, num_subcores=NS
)


_TGRP = 16


def _tr_body(x_ref, o_ref):
    xb = x_ref[...]
    o_ref[...] = (
        xb.reshape(H, _TGRP, RPG).transpose(1, 0, 2).reshape(_TGRP * H, RPG)
    )


_transpose = pl.pallas_call(
    _tr_body,
    grid=(G // _TGRP,),
    in_specs=[pl.BlockSpec((H, _TGRP * RPG), lambda i: (0, i))],
    out_specs=pl.BlockSpec((_TGRP * H, RPG), lambda i: (i, 0)),
    out_shape=jax.ShapeDtypeStruct((G * H, RPG), jnp.int32),
)

V = 1000000
_CW = 1024
_NFULL = 975


@functools.partial(
    pl.kernel,
    out_type=jax.ShapeDtypeStruct((V // 4, 128), jnp.float32),
    mesh=_mesh,
    scratch_types=[
        pltpu.VMEM((E, _CW), jnp.float32),
        pltpu.VMEM((_CW // 4, 128), jnp.float32),
        pltpu.SemaphoreType.DMA,
    ],
    compiler_params=pltpu.CompilerParams(
        use_tc_tiling_on_sc=True, needs_layout_passes=False
    ),
)
def _sc_repack(embT, tailT, out, sv, ob, sem):
    wid = lax.axis_index("s") * NC + lax.axis_index("c")
    iota = jnp.arange(16, dtype=jnp.int32)
    e_lo = iota
    e_hi = iota + 16

    def do_chunk(src, soff, dnv, pnv, q0):
        cps = [
            pltpu.async_copy(
                src.at[pl.ds(b * 8, 8), pl.ds(soff, dnv)],
                sv.at[pl.ds(b * 8, 8), pl.ds(0, dnv)],
                sem,
            )
            for b in range(4)
        ]
        for cp in cps:
            cp.wait()

        def vbody(vp, carry):
            vcol = jnp.full((16,), vp, jnp.int32)
            x0 = plsc.load_gather(sv, [e_lo, vcol])
            x1 = plsc.load_gather(sv, [e_hi, vcol])
            q = jnp.full((16,), vp >> 2, jnp.int32)
            lane = (vp & 3) * 32 + iota
            plsc.store_scatter(ob, [q, lane], x0)
            plsc.store_scatter(ob, [q, lane + 16], x1)
            return carry

        lax.fori_loop(0, pnv, vbody, 0)
        pltpu.sync_copy(ob.at[pl.ds(0, pnv // 4)], out.at[pl.ds(q0, pnv // 4)])

    start = (_NFULL * wid) // NW
    stop = (_NFULL * (wid + 1)) // NW

    def chunk_body(i, carry):
        do_chunk(
            embT,
            pl.multiple_of(i * _CW, _CW),
            _CW,
            _CW,
            pl.multiple_of(i * (_CW // 4), 8),
        )
        return carry

    lax.fori_loop(start, stop, chunk_body, 0)

    @pl.when(wid == NW - 1)
    def _tail():
        do_chunk(embT, 998400, _CW, _CW, 249600)
        do_chunk(embT, 999424, 512, 512, 249856)
        do_chunk(tailT, 0, 128, 64, 249984)


@functools.partial(
    pl.kernel,
    out_type=jax.ShapeDtypeStruct((B, E), jnp.float32),
    mesh=_mesh,
    scratch_types=[
        pltpu.VMEM((2, H, RPG), jnp.int32),
        pltpu.VMEM((2, RPG, E), jnp.float32),
        pltpu.SemaphoreType.DMA,
        pltpu.SemaphoreType.DMA,
    ],
    compiler_params=pltpu.CompilerParams(use_tc_tiling_on_sc=False),
)
def _sc_pool(xt, table, out, idx_v, acc, sem0, sem1):
    wid = lax.axis_index("s") * NC + lax.axis_index("c")
    zero = jnp.zeros((16,), jnp.float32)
    sems = (sem0, sem1)

    def prep(g):
        b = g % 2
        gg = wid * GPW + g
        pltpu.sync_copy(xt.at[pl.ds(gg * H, H)], idx_v.at[b])

        def zero_body(r, carry):
            acc[b, r, pl.ds(0, 16)] = zero
            acc[b, r, pl.ds(16, 16)] = zero
            return carry

        lax.fori_loop(0, RPG, zero_body, 0, unroll=8)

        def fire_body(h, carry):
            pltpu.async_copy(
                table.at[idx_v.at[b, h]], acc.at[b], sems[b], add=True
            )
            return carry

        lax.fori_loop(0, H, fire_body, 0)

    prep(0)
    for g in range(GPW):
        b = g % 2
        if g + 1 < GPW:
            prep(g + 1)

        def drain_body(h, carry):
            pltpu.make_async_copy(
                table.at[idx_v.at[b, 0]], acc.at[b], sems[b]
            ).wait()
            return carry

        lax.fori_loop(0, H, drain_body, 0)
        gg = wid * GPW + g
        pltpu.sync_copy(acc.at[b], out.at[pl.ds(gg * RPG, RPG)])


def _mlp_body(p_ref, w1_ref, b1_ref, w2_ref, b2_ref, o_ref):
    h = jnp.dot(p_ref[...], w1_ref[...], preferred_element_type=jnp.float32)
    h = h + b1_ref[...]
    o = jnp.dot(h, w2_ref[...], preferred_element_type=jnp.float32)
    o_ref[...] = o + b2_ref[...]


_MLP_BLOCK = 2048
_mlp = pl.pallas_call(
    _mlp_body,
    grid=(B // _MLP_BLOCK,),
    in_specs=[
        pl.BlockSpec((_MLP_BLOCK, E), lambda i: (i, 0)),
        pl.BlockSpec((E, HID), lambda i: (0, 0)),
        pl.BlockSpec((1, HID), lambda i: (0, 0)),
        pl.BlockSpec((HID, OUT), lambda i: (0, 0)),
        pl.BlockSpec((1, OUT), lambda i: (0, 0)),
    ],
    out_specs=pl.BlockSpec((_MLP_BLOCK, OUT), lambda i: (i, 0)),
    out_shape=jax.ShapeDtypeStruct((B, OUT), jnp.float32),
)


@jax.jit
def kernel(x, embeddings, W1, b1, W2, b2):
    xt = _transpose(x.astype(jnp.int32).T)
    embT = embeddings.T
    tailT = jnp.pad(lax.slice(embT, (0, 999936), (E, V)), ((0, 0), (0, 64)))
    table_lin = _sc_repack(embT, tailT).reshape(V, E)
    pooled = _sc_pool(xt, table_lin)
    return _mlp(pooled, W1, b1.reshape(1, HID), W2, b2.reshape(1, OUT))


# speedup vs baseline: 1.7663x; 1.7663x over previous
import functools

import jax
import jax.numpy as jnp
from jax import lax
from jax.experimental import pallas as pl
from jax.experimental.pallas import tpu as pltpu
from jax.experimental.pallas import tpu_sc as plsc

B = 16384
H = 200
E = 32
HID = 128
OUT = 2

NC, NS = 2, 16
NW = NC * NS
RPG = 128
G = B // RPG
GPW = G // NW
IPG = RPG * H

_mesh = plsc.VectorSubcoreMesh(
    core_axis_name="c", subcore_axis_name="s", num_cores=NC, num_subcores=NS
)


_TGRP = 16


def _tr_body(x_ref, o_ref):
    xb = x_ref[...]
    o_ref[...] = (
        xb.reshape(H, _TGRP, RPG).transpose(1, 0, 2).reshape(_TGRP * H, RPG)
    )


_transpose = pl.pallas_call(
    _tr_body,
    grid=(G // _TGRP,),
    in_specs=[pl.BlockSpec((H, _TGRP * RPG), lambda i: (0, i))],
    out_specs=pl.BlockSpec((_TGRP * H, RPG), lambda i: (i, 0)),
    out_shape=jax.ShapeDtypeStruct((G * H, RPG), jnp.int32),
)

V = 1000000
_RB = 16384
_RGRID = (V + _RB - 1) // _RB


def _repack_body(t_ref, o_ref):
    t2 = t_ref[...].T.reshape(_RB // 4, 4, E)
    o_ref[...] = jnp.concatenate(
        [t2[:, 0, :], t2[:, 1, :], t2[:, 2, :], t2[:, 3, :]], axis=1
    )


_repack = pl.pallas_call(
    _repack_body,
    grid=(_RGRID,),
    in_specs=[pl.BlockSpec((E, _RB), lambda i: (0, i))],
    out_specs=pl.BlockSpec((_RB // 4, 128), lambda i: (i, 0)),
    out_shape=jax.ShapeDtypeStruct((V // 4, 128), jnp.float32),
)


@functools.partial(
    pl.kernel,
    out_type=jax.ShapeDtypeStruct((B, E), jnp.float32),
    mesh=_mesh,
    scratch_types=[
        pltpu.VMEM((2, H, RPG), jnp.int32),
        pltpu.VMEM((2, RPG, E), jnp.float32),
        pltpu.SemaphoreType.DMA,
        pltpu.SemaphoreType.DMA,
    ],
    compiler_params=pltpu.CompilerParams(use_tc_tiling_on_sc=False),
)
def _sc_pool(xt, table, out, idx_v, acc, sem0, sem1):
    wid = lax.axis_index("s") * NC + lax.axis_index("c")
    zero = jnp.zeros((16,), jnp.float32)
    sems = (sem0, sem1)

    def prep(g):
        b = g % 2
        gg = wid * GPW + g
        pltpu.sync_copy(xt.at[pl.ds(gg * H, H)], idx_v.at[b])

        def zero_body(r, carry):
            acc[b, r, pl.ds(0, 16)] = zero
            acc[b, r, pl.ds(16, 16)] = zero
            return carry

        lax.fori_loop(0, RPG, zero_body, 0, unroll=8)

        def fire_body(h, carry):
            pltpu.async_copy(
                table.at[idx_v.at[b, h]], acc.at[b], sems[b], add=True
            )
            return carry

        lax.fori_loop(0, H, fire_body, 0)

    prep(0)
    for g in range(GPW):
        b = g % 2
        if g + 1 < GPW:
            prep(g + 1)

        def drain_body(h, carry):
            pltpu.make_async_copy(
                table.at[idx_v.at[b, 0]], acc.at[b], sems[b]
            ).wait()
            return carry

        lax.fori_loop(0, H, drain_body, 0)
        gg = wid * GPW + g
        pltpu.sync_copy(acc.at[b], out.at[pl.ds(gg * RPG, RPG)])


def _mlp_body(p_ref, w1_ref, b1_ref, w2_ref, b2_ref, o_ref):
    h = jnp.dot(p_ref[...], w1_ref[...], preferred_element_type=jnp.float32)
    h = h + b1_ref[...]
    o = jnp.dot(h, w2_ref[...], preferred_element_type=jnp.float32)
    o_ref[...] = o + b2_ref[...]


_MLP_BLOCK = 2048
_mlp = pl.pallas_call(
    _mlp_body,
    grid=(B // _MLP_BLOCK,),
    in_specs=[
        pl.BlockSpec((_MLP_BLOCK, E), lambda i: (i, 0)),
        pl.BlockSpec((E, HID), lambda i: (0, 0)),
        pl.BlockSpec((1, HID), lambda i: (0, 0)),
        pl.BlockSpec((HID, OUT), lambda i: (0, 0)),
        pl.BlockSpec((1, OUT), lambda i: (0, 0)),
    ],
    out_specs=pl.BlockSpec((_MLP_BLOCK, OUT), lambda i: (i, 0)),
    out_shape=jax.ShapeDtypeStruct((B, OUT), jnp.float32),
)


@jax.jit
def kernel(x, embeddings, W1, b1, W2, b2):
    xt = _transpose(x.astype(jnp.int32).T)
    table_lin = _repack(embeddings.T).reshape(V, E)
    pooled = _sc_pool(xt, table_lin)
    return _mlp(pooled, W1, b1.reshape(1, HID), W2, b2.reshape(1, OUT))
